# baseline (speedup 1.0000x reference)
import functools

import jax
import jax.numpy as jnp
from jax import lax
from jax.experimental import pallas as pl
from jax.experimental.pallas import tpu as pltpu
from jax.experimental.pallas import tpu_sc as plsc

H = 128
HH = 64
HW = 80
NC, NS, L = 2, 16, 16
NW = NC * NS
EK = 80


def _sc_mesh():
    return plsc.VectorSubcoreMesh(core_axis_name="c", subcore_axis_name="s")


def _edge_sc(h2, as_v, ad_v, src3d, dst3d):
    n_src = h2.shape[1]
    n_dst = ad_v.shape[0]
    nch, cb = src3d.shape[1], src3d.shape[2]
    rpt = 640
    qmax = -(-max(rpt, n_dst - 15 * rpt) // EK)

    @functools.partial(
        pl.kernel,
        mesh=_sc_mesh(),
        compiler_params=pltpu.CompilerParams(
            needs_layout_passes=False, use_tc_tiling_on_sc=False),
        out_type=jax.ShapeDtypeStruct((NC, n_dst, HW), jnp.float32),
        scratch_types=[
            pltpu.VMEM((n_src,), jnp.float32),
            pltpu.VMEM((n_dst,), jnp.float32),
            pltpu.VMEM((cb, EK), jnp.int32),
            pltpu.VMEM((cb, EK), jnp.int32),
            pltpu.VMEM((2, EK), jnp.float32),
            pltpu.VMEM((2, EK, HH), jnp.float32),
            pltpu.VMEM((2, EK, HW), jnp.float32),
            pltpu.VMEM_SHARED((n_dst, HW), jnp.float32),
            pltpu.SemaphoreType.DMA,
            pltpu.SemaphoreType.DMA,
            pltpu.SemaphoreType.DMA,
            pltpu.SemaphoreType.DMA,
        ],
    )
    def body(h_hbm, as_hbm, ad_hbm, src_hbm, dst_hbm, out_hbm,
             as_t, ad_t, src_t, dst_t, e_2, rows_g2, rows_s2, u_sh,
             sg0, sg1, ss0, ss1):
        ci = lax.axis_index("c")
        si = lax.axis_index("s")

        rstart = si * rpt
        rcnt = jnp.minimum(n_dst - rstart, rpt)

        pltpu.sync_copy(as_hbm, as_t)
        pltpu.sync_copy(ad_hbm, ad_t)

        z_t = rows_s2.at[0]

        def zrow(r, _):
            for cc in range(HW // L):
                z_t[r, pl.ds(cc * L, L)] = jnp.zeros((L,), jnp.float32)
            return 0
        lax.fori_loop(0, EK, zrow, 0)
        for q in range(qmax):
            @pl.when(q * EK < rcnt)
            def _():
                pltpu.sync_copy(z_t, u_sh.at[pl.ds(rstart + q * EK, EK)])

        def mx(t_ref, n):
            def step(i, mv):
                return jnp.maximum(mv, t_ref[pl.ds(i * L, L)])
            return lax.fori_loop(0, n // L, step,
                                 jnp.full((L,), -3e38, jnp.float32))

        def lane_max(v):
            tmp = e_2.at[0]
            for sh in (8, 4, 2, 1):
                tmp[pl.ds(0, L)] = v
                idx = (lax.iota(jnp.int32, L) + sh) & (L - 1)
                v = jnp.maximum(v, plsc.load_gather(tmp, [idx]))
            return v
        cmax = lane_max(mx(as_t, n_src)) + lane_max(mx(ad_t, n_dst))
        cmax = jnp.where(cmax >= 0, cmax, 0.2 * cmax)

        plsc.subcore_barrier()

        lane0 = lax.iota(jnp.int32, L) == 0
        slots = ((rows_g2.at[0], rows_s2.at[0], e_2.at[0], sg0, ss0),
                 (rows_g2.at[1], rows_s2.at[1], e_2.at[1], sg1, ss1))

        def fire_gather(j, b):
            pltpu.async_copy(h_hbm.at[ci].at[src_t.at[j]], slots[b][0],
                             slots[b][3])

        def step2(jj, _):
            j0 = 2 * jj
            for b in range(2):
                j = j0 + b
                rg, rs, et, sg, ss = slots[b]

                @pl.when(j + 1 < cb)
                def _():
                    fire_gather(j + 1, 1 - b)
                for k in range(EK // L):
                    isv = src_t[j, pl.ds(k * L, L)]
                    idv = dst_t[j, pl.ds(k * L, L)]
                    av = (plsc.load_gather(as_t, [isv])
                          + plsc.load_gather(ad_t, [idv]))
                    av = jnp.where(av >= 0, av, 0.2 * av)
                    et[pl.ds(k * L, L)] = jnp.exp(av - cmax)
                pltpu.make_async_copy(
                    h_hbm.at[ci].at[src_t.at[j]], rg, sg).wait()
                @pl.when(j >= 2)
                def _():
                    pltpu.make_async_copy(
                        rs, u_sh.at[dst_t.at[j - 2]], ss).wait()

                @plsc.parallel_loop(0, EK, 1, unroll=16)
                def _(r):
                    ebc = plsc.load_gather(
                        et, [jnp.full((L,), r, jnp.int32)])
                    for cc in range(HH // L):
                        rs[r, pl.ds(cc * L, L)] = (
                            rg[r, pl.ds(cc * L, L)] * ebc)
                    rs[r, pl.ds(HH, L)] = jnp.where(lane0, ebc, 0.0)
                pltpu.async_copy(rs, u_sh.at[dst_t.at[j]], ss, add=True)
            return 0

        def chunk(ch, _):
            pltpu.sync_copy(src_hbm.at[si, ch], src_t)
            pltpu.sync_copy(dst_hbm.at[si, ch], dst_t)
            fire_gather(0, 0)
            lax.fori_loop(0, cb // 2, step2, 0)
            for b in range(2):
                pltpu.make_async_copy(
                    slots[b][1], u_sh.at[dst_t.at[cb - 2 + b]],
                    slots[b][4]).wait()
            return 0
        lax.fori_loop(0, nch, chunk, 0)

        plsc.subcore_barrier()

        for q in range(qmax):
            @pl.when(q * EK < rcnt)
            def _():
                pltpu.sync_copy(u_sh.at[pl.ds(rstart + q * EK, EK)], z_t)
                pltpu.sync_copy(
                    z_t, out_hbm.at[ci, pl.ds(rstart + q * EK, EK)])

    return body(h2, as_v, ad_v, src3d, dst3d)


def _pair_gather_sc(h_m, h_p, mi3, pi3):
    n, gb, gk = h_m.shape[0], mi3.shape[1], mi3.shape[2]
    bpt = gb * gk
    b_tot = NW * bpt

    @functools.partial(
        pl.kernel,
        mesh=_sc_mesh(),
        compiler_params=pltpu.CompilerParams(
            needs_layout_passes=False, use_tc_tiling_on_sc=False),
        out_type=[jax.ShapeDtypeStruct((b_tot, H), jnp.float32),
                  jax.ShapeDtypeStruct((b_tot, H), jnp.float32)],
        scratch_types=[
            pltpu.VMEM((gb, gk), jnp.int32),
            pltpu.VMEM((2, gk, H), jnp.float32),
            pltpu.SemaphoreType.DMA,
            pltpu.SemaphoreType.DMA,
        ],
    )
    def gbody(hm_hbm, hp_hbm, mi_hbm, pi_hbm, om_hbm, op_hbm,
              idx_t, rows2, s0, s1):
        ci = lax.axis_index("c")
        si = lax.axis_index("s")
        wid = ci * NS + si
        base = wid * bpt
        sems = (s0, s1)

        for tab, idx_hbm, out in ((hm_hbm, mi_hbm, om_hbm),
                                  (hp_hbm, pi_hbm, op_hbm)):
            pltpu.sync_copy(idx_hbm.at[wid], idx_t)
            pltpu.async_copy(tab.at[idx_t.at[0]], rows2.at[0], s0)

            def gstep(jj, _):
                for b in range(2):
                    j = 2 * jj + b

                    @pl.when(j + 1 < gb)
                    def _():
                        pltpu.async_copy(tab.at[idx_t.at[j + 1]],
                                         rows2.at[1 - b], sems[1 - b])
                    pltpu.make_async_copy(
                        tab.at[idx_t.at[j]], rows2.at[b], sems[b]).wait()
                    pltpu.sync_copy(rows2.at[b],
                                    out.at[pl.ds(base + j * gk, gk)])
                return 0
            lax.fori_loop(0, gb // 2, gstep, 0)

    return gbody(h_m, h_p, mi3, pi3)


def _mm_body(x_ref, w_ref, b_ref, o_ref):
    o_ref[...] = jnp.dot(x_ref[...], w_ref[...],
                         preferred_element_type=jnp.float32) + b_ref[...]


def _matmul(x, w, b, bm=1000):
    m, k = x.shape
    n = w.shape[1]
    return pl.pallas_call(
        _mm_body,
        grid=(m // bm,),
        in_specs=[
            pl.BlockSpec((bm, k), lambda i: (i, 0)),
            pl.BlockSpec((k, n), lambda i: (0, 0)),
            pl.BlockSpec((1, n), lambda i: (0, 0)),
        ],
        out_specs=pl.BlockSpec((bm, n), lambda i: (i, 0)),
        out_shape=jax.ShapeDtypeStruct((m, n), jnp.float32),
    )(x, w, b.reshape(1, n))


def _combine_body(ua_ref, ub_ref, ba_ref, bb_ref, g_ref, b_ref, o_ref):
    def gat(u_ref, bias):
        num = jnp.concatenate([u_ref[0, :, :HH], u_ref[1, :, :HH]], axis=1)
        den = u_ref[0, :, HH:HH + 1]
        return num / (den + 1e-16) + bias
    o = 0.5 * (gat(ua_ref, ba_ref[...]) + gat(ub_ref, bb_ref[...]))
    r = jnp.maximum(o, 0.0)
    mu = jnp.mean(r, axis=0, keepdims=True)
    var = jnp.mean((r - mu) ** 2, axis=0, keepdims=True)
    o_ref[...] = ((r - mu) / jnp.sqrt(var + 1e-5)) * g_ref[...] + b_ref[...]


def _combine_bn(u_a, u_b, bias_a, bias_b, g, b):
    n = u_a.shape[1]
    full = lambda s: pl.BlockSpec(s, lambda: tuple(0 for _ in s))
    return pl.pallas_call(
        _combine_body,
        in_specs=[full(u_a.shape), full(u_b.shape),
                  full((1, H)), full((1, H)), full((1, H)), full((1, H))],
        out_specs=full((n, H)),
        out_shape=jax.ShapeDtypeStruct((n, H), jnp.float32),
    )(u_a, u_b, bias_a.reshape(1, H), bias_b.reshape(1, H),
      g.reshape(1, H), b.reshape(1, H))


def _mlp_body(m_ref, p_ref, w1_ref, b1_ref, w2_ref, b2_ref, o_ref):
    m = m_ref[...]
    p = p_ref[...]
    e = jnp.concatenate([m, p, m * p, jnp.abs(m - p)], axis=1)
    h = jnp.maximum(
        jnp.dot(e, w1_ref[...], preferred_element_type=jnp.float32) + b1_ref[...],
        0.0,
    )
    y = jnp.sum(h * w2_ref[...], axis=1) + b2_ref[0, 0]
    o_ref[...] = y.reshape(o_ref.shape)


def _mlp_head(m_emb, p_emb, mlp):
    B = m_emb.shape[0]
    BM = 2048
    grid = B // BM
    w1 = mlp['W1']
    b1 = mlp['b1'].reshape(1, -1)
    w2 = mlp['W2'].reshape(1, -1)
    b2 = mlp['b2'].reshape(1, 1)
    out = pl.pallas_call(
        _mlp_body,
        grid=(grid,),
        in_specs=[
            pl.BlockSpec((BM, H), lambda i: (i, 0)),
            pl.BlockSpec((BM, H), lambda i: (i, 0)),
            pl.BlockSpec(w1.shape, lambda i: (0, 0)),
            pl.BlockSpec(b1.shape, lambda i: (0, 0)),
            pl.BlockSpec(w2.shape, lambda i: (0, 0)),
            pl.BlockSpec(b2.shape, lambda i: (0, 0)),
        ],
        out_specs=pl.BlockSpec((BM // 128, 128), lambda i: (i, 0)),
        out_shape=jax.ShapeDtypeStruct((B // 128, 128), jnp.float32),
    )(m_emb, p_emb, w1, b1, w2, b2)
    return out.reshape(B)


def _edge_u(h_full, as_v, ad_v, ei):
    h2 = jnp.stack([h_full[:, :HH], h_full[:, HH:]])
    src3d = ei[0].reshape(NS, 5, -1, EK)
    dst3d = ei[1].reshape(NS, 5, -1, EK)
    return _edge_sc(h2, as_v, ad_v, src3d, dst3d)


def _wcat(pa, pb, pda, pdb):
    cols = jnp.stack([
        pa['W_src'] @ pa['att_src'],
        pb['W_src'] @ pb['att_src'],
        pda['W_dst'] @ pda['att_dst'],
        pdb['W_dst'] @ pdb['att_dst'],
    ], axis=1)
    pad = jnp.zeros((H, 124), jnp.float32)
    return jnp.concatenate([pa['W_src'], pb['W_src'], cols, pad], axis=1)


def kernel(x_metabolite, x_protein, params, edge_index_pp, edge_index_mm,
           edge_index_mp, edge_index_pm, metabolite_idx, protein_idx):
    h_m = _matmul(x_metabolite, params['proj_m']['W'], params['proj_m']['b'])
    h_p = _matmul(x_protein, params['proj_p']['W'], params['proj_p']['b'])
    z384 = jnp.zeros((384,), jnp.float32)
    for lp in params['layers']:
        y_p = _matmul(h_p, _wcat(lp['pp'], lp['pm'], lp['pp'], lp['mp']), z384)
        y_m = _matmul(h_m, _wcat(lp['mm'], lp['mp'], lp['mm'], lp['pm']), z384)
        u_pp = _edge_u(y_p[:, :H], y_p[:, 256], y_p[:, 258], edge_index_pp)
        u_mp = _edge_u(y_m[:, H:2 * H], y_m[:, 257], y_p[:, 259], edge_index_mp)
        u_mm = _edge_u(y_m[:, :H], y_m[:, 256], y_m[:, 258], edge_index_mm)
        u_pm = _edge_u(y_p[:, H:2 * H], y_p[:, 257], y_m[:, 259], edge_index_pm)
        h_p = _combine_bn(u_pp, u_mp, lp['pp']['bias'], lp['mp']['bias'],
                          lp['bn_p']['g'], lp['bn_p']['b'])
        h_m = _combine_bn(u_mm, u_pm, lp['mm']['bias'], lp['pm']['bias'],
                          lp['bn_m']['g'], lp['bn_m']['b'])
    mi3 = metabolite_idx.reshape(NW, -1, 128)
    pi3 = protein_idx.reshape(NW, -1, 128)
    m_emb, p_emb = _pair_gather_sc(h_m, h_p, mi3, pi3)
    return _mlp_head(m_emb, p_emb, params['mlp'])


# speedup vs baseline: 1.0299x; 1.0299x over previous
import functools

import jax
import jax.numpy as jnp
from jax import lax
from jax.experimental import pallas as pl
from jax.experimental.pallas import tpu as pltpu
from jax.experimental.pallas import tpu_sc as plsc

H = 128
HH = 64
HW = 80
NC, NS, L = 2, 16, 16
NW = NC * NS
EK = 80


def _sc_mesh():
    return plsc.VectorSubcoreMesh(core_axis_name="c", subcore_axis_name="s")


def _edge_sc(h2, as_v, ad_v, src3d, dst3d):
    n_src = h2.shape[1]
    n_dst = ad_v.shape[0]
    nch, cb = src3d.shape[1], src3d.shape[2]
    rpt = 640
    qmax = -(-max(rpt, n_dst - 15 * rpt) // EK)

    @functools.partial(
        pl.kernel,
        mesh=_sc_mesh(),
        compiler_params=pltpu.CompilerParams(
            needs_layout_passes=False, use_tc_tiling_on_sc=False),
        out_type=jax.ShapeDtypeStruct((NC, n_dst, HW), jnp.float32),
        scratch_types=[
            pltpu.VMEM((n_src,), jnp.float32),
            pltpu.VMEM((n_dst,), jnp.float32),
            pltpu.VMEM((cb, EK), jnp.int32),
            pltpu.VMEM((cb, EK), jnp.int32),
            pltpu.VMEM((2, EK), jnp.float32),
            pltpu.VMEM((2, EK, HH), jnp.float32),
            pltpu.VMEM((2, EK, HW), jnp.float32),
            pltpu.VMEM_SHARED((n_dst, HW), jnp.float32),
            pltpu.SemaphoreType.DMA,
            pltpu.SemaphoreType.DMA,
            pltpu.SemaphoreType.DMA,
            pltpu.SemaphoreType.DMA,
        ],
    )
    def body(h_hbm, as_hbm, ad_hbm, src_hbm, dst_hbm, out_hbm,
             as_t, ad_t, src_t, dst_t, e_2, rows_g2, rows_s2, u_sh,
             sg0, sg1, ss0, ss1):
        ci = lax.axis_index("c")
        si = lax.axis_index("s")

        rstart = si * rpt
        rcnt = jnp.minimum(n_dst - rstart, rpt)

        pltpu.sync_copy(as_hbm, as_t)
        pltpu.sync_copy(ad_hbm, ad_t)

        z_t = rows_s2.at[0]

        def zrow(r, _):
            for cc in range(HW // L):
                z_t[r, pl.ds(cc * L, L)] = jnp.zeros((L,), jnp.float32)
            return 0
        lax.fori_loop(0, EK, zrow, 0)
        for q in range(qmax):
            @pl.when(q * EK < rcnt)
            def _():
                pltpu.sync_copy(z_t, u_sh.at[pl.ds(rstart + q * EK, EK)])

        def mx(t_ref, n):
            nacc = 8
            stride = nacc * L
            init = [jnp.full((L,), -3e38, jnp.float32)] * nacc

            def step(i, mvs):
                return [jnp.maximum(mvs[k], t_ref[pl.ds(i * stride + k * L, L)])
                        for k in range(nacc)]
            mvs = lax.fori_loop(0, n // stride, step, init)
            mv = mvs[0]
            for k in range(1, nacc):
                mv = jnp.maximum(mv, mvs[k])
            for t in range((n % stride) // L):
                mv = jnp.maximum(
                    mv, t_ref[pl.ds((n // stride) * stride + t * L, L)])
            return mv

        def lane_max(v):
            tmp = e_2.at[0]
            for sh in (8, 4, 2, 1):
                tmp[pl.ds(0, L)] = v
                idx = (lax.iota(jnp.int32, L) + sh) & (L - 1)
                v = jnp.maximum(v, plsc.load_gather(tmp, [idx]))
            return v
        cmax = lane_max(mx(as_t, n_src)) + lane_max(mx(ad_t, n_dst))
        cmax = jnp.where(cmax >= 0, cmax, 0.2 * cmax)

        plsc.subcore_barrier()

        lane0 = lax.iota(jnp.int32, L) == 0
        slots = ((rows_g2.at[0], rows_s2.at[0], e_2.at[0], sg0, ss0),
                 (rows_g2.at[1], rows_s2.at[1], e_2.at[1], sg1, ss1))

        def fire_gather(j, b):
            pltpu.async_copy(h_hbm.at[ci].at[src_t.at[j]], slots[b][0],
                             slots[b][3])

        def step2(jj, _):
            j0 = 2 * jj
            for b in range(2):
                j = j0 + b
                rg, rs, et, sg, ss = slots[b]

                @pl.when(j + 1 < cb)
                def _():
                    fire_gather(j + 1, 1 - b)
                for k in range(EK // L):
                    isv = src_t[j, pl.ds(k * L, L)]
                    idv = dst_t[j, pl.ds(k * L, L)]
                    av = (plsc.load_gather(as_t, [isv])
                          + plsc.load_gather(ad_t, [idv]))
                    av = jnp.where(av >= 0, av, 0.2 * av)
                    et[pl.ds(k * L, L)] = jnp.exp(av - cmax)
                pltpu.make_async_copy(
                    h_hbm.at[ci].at[src_t.at[j]], rg, sg).wait()
                @pl.when(j >= 2)
                def _():
                    pltpu.make_async_copy(
                        rs, u_sh.at[dst_t.at[j - 2]], ss).wait()

                @plsc.parallel_loop(0, EK, 1, unroll=8)
                def _(r):
                    ebc = plsc.load_gather(
                        et, [jnp.full((L,), r, jnp.int32)])
                    for cc in range(HH // L):
                        rs[r, pl.ds(cc * L, L)] = (
                            rg[r, pl.ds(cc * L, L)] * ebc)
                    rs[r, pl.ds(HH, L)] = jnp.where(lane0, ebc, 0.0)
                pltpu.async_copy(rs, u_sh.at[dst_t.at[j]], ss, add=True)
            return 0

        def chunk(ch, _):
            pltpu.sync_copy(src_hbm.at[si, ch], src_t)
            pltpu.sync_copy(dst_hbm.at[si, ch], dst_t)
            fire_gather(0, 0)
            lax.fori_loop(0, cb // 2, step2, 0)
            for b in range(2):
                pltpu.make_async_copy(
                    slots[b][1], u_sh.at[dst_t.at[cb - 2 + b]],
                    slots[b][4]).wait()
            return 0
        lax.fori_loop(0, nch, chunk, 0)

        plsc.subcore_barrier()

        for q in range(qmax):
            @pl.when(q * EK < rcnt)
            def _():
                pltpu.sync_copy(u_sh.at[pl.ds(rstart + q * EK, EK)], z_t)
                pltpu.sync_copy(
                    z_t, out_hbm.at[ci, pl.ds(rstart + q * EK, EK)])

    return body(h2, as_v, ad_v, src3d, dst3d)


def _pair_gather_sc(h_m, h_p, mi3, pi3):
    n, gb, gk = h_m.shape[0], mi3.shape[1], mi3.shape[2]
    bpt = gb * gk
    b_tot = NW * bpt

    @functools.partial(
        pl.kernel,
        mesh=_sc_mesh(),
        compiler_params=pltpu.CompilerParams(
            needs_layout_passes=False, use_tc_tiling_on_sc=False),
        out_type=[jax.ShapeDtypeStruct((b_tot, H), jnp.float32),
                  jax.ShapeDtypeStruct((b_tot, H), jnp.float32)],
        scratch_types=[
            pltpu.VMEM((gb, gk), jnp.int32),
            pltpu.VMEM((2, gk, H), jnp.float32),
            pltpu.SemaphoreType.DMA,
            pltpu.SemaphoreType.DMA,
        ],
    )
    def gbody(hm_hbm, hp_hbm, mi_hbm, pi_hbm, om_hbm, op_hbm,
              idx_t, rows2, s0, s1):
        ci = lax.axis_index("c")
        si = lax.axis_index("s")
        wid = ci * NS + si
        base = wid * bpt
        sems = (s0, s1)

        for tab, idx_hbm, out in ((hm_hbm, mi_hbm, om_hbm),
                                  (hp_hbm, pi_hbm, op_hbm)):
            pltpu.sync_copy(idx_hbm.at[wid], idx_t)
            pltpu.async_copy(tab.at[idx_t.at[0]], rows2.at[0], s0)

            def gstep(jj, _):
                for b in range(2):
                    j = 2 * jj + b

                    @pl.when(j + 1 < gb)
                    def _():
                        pltpu.async_copy(tab.at[idx_t.at[j + 1]],
                                         rows2.at[1 - b], sems[1 - b])
                    pltpu.make_async_copy(
                        tab.at[idx_t.at[j]], rows2.at[b], sems[b]).wait()
                    pltpu.sync_copy(rows2.at[b],
                                    out.at[pl.ds(base + j * gk, gk)])
                return 0
            lax.fori_loop(0, gb // 2, gstep, 0)

    return gbody(h_m, h_p, mi3, pi3)


def _mm_body(x_ref, w_ref, b_ref, o_ref):
    o_ref[...] = jnp.dot(x_ref[...], w_ref[...],
                         preferred_element_type=jnp.float32) + b_ref[...]


def _matmul(x, w, b, bm=1000):
    m, k = x.shape
    n = w.shape[1]
    return pl.pallas_call(
        _mm_body,
        grid=(m // bm,),
        in_specs=[
            pl.BlockSpec((bm, k), lambda i: (i, 0)),
            pl.BlockSpec((k, n), lambda i: (0, 0)),
            pl.BlockSpec((1, n), lambda i: (0, 0)),
        ],
        out_specs=pl.BlockSpec((bm, n), lambda i: (i, 0)),
        out_shape=jax.ShapeDtypeStruct((m, n), jnp.float32),
    )(x, w, b.reshape(1, n))


def _combine_body(ua_ref, ub_ref, ba_ref, bb_ref, g_ref, b_ref, o_ref):
    def gat(u_ref, bias):
        num = jnp.concatenate([u_ref[0, :, :HH], u_ref[1, :, :HH]], axis=1)
        den = u_ref[0, :, HH:HH + 1]
        return num / (den + 1e-16) + bias
    o = 0.5 * (gat(ua_ref, ba_ref[...]) + gat(ub_ref, bb_ref[...]))
    r = jnp.maximum(o, 0.0)
    mu = jnp.mean(r, axis=0, keepdims=True)
    var = jnp.mean((r - mu) ** 2, axis=0, keepdims=True)
    o_ref[...] = ((r - mu) / jnp.sqrt(var + 1e-5)) * g_ref[...] + b_ref[...]


def _combine_bn(u_a, u_b, bias_a, bias_b, g, b):
    n = u_a.shape[1]
    full = lambda s: pl.BlockSpec(s, lambda: tuple(0 for _ in s))
    return pl.pallas_call(
        _combine_body,
        in_specs=[full(u_a.shape), full(u_b.shape),
                  full((1, H)), full((1, H)), full((1, H)), full((1, H))],
        out_specs=full((n, H)),
        out_shape=jax.ShapeDtypeStruct((n, H), jnp.float32),
    )(u_a, u_b, bias_a.reshape(1, H), bias_b.reshape(1, H),
      g.reshape(1, H), b.reshape(1, H))


def _mlp_body(m_ref, p_ref, w1_ref, b1_ref, w2_ref, b2_ref, o_ref):
    m = m_ref[...]
    p = p_ref[...]
    e = jnp.concatenate([m, p, m * p, jnp.abs(m - p)], axis=1)
    h = jnp.maximum(
        jnp.dot(e, w1_ref[...], preferred_element_type=jnp.float32) + b1_ref[...],
        0.0,
    )
    y = jnp.sum(h * w2_ref[...], axis=1) + b2_ref[0, 0]
    o_ref[...] = y.reshape(o_ref.shape)


def _mlp_head(m_emb, p_emb, mlp):
    B = m_emb.shape[0]
    BM = 2048
    grid = B // BM
    w1 = mlp['W1']
    b1 = mlp['b1'].reshape(1, -1)
    w2 = mlp['W2'].reshape(1, -1)
    b2 = mlp['b2'].reshape(1, 1)
    out = pl.pallas_call(
        _mlp_body,
        grid=(grid,),
        in_specs=[
            pl.BlockSpec((BM, H), lambda i: (i, 0)),
            pl.BlockSpec((BM, H), lambda i: (i, 0)),
            pl.BlockSpec(w1.shape, lambda i: (0, 0)),
            pl.BlockSpec(b1.shape, lambda i: (0, 0)),
            pl.BlockSpec(w2.shape, lambda i: (0, 0)),
            pl.BlockSpec(b2.shape, lambda i: (0, 0)),
        ],
        out_specs=pl.BlockSpec((BM // 128, 128), lambda i: (i, 0)),
        out_shape=jax.ShapeDtypeStruct((B // 128, 128), jnp.float32),
    )(m_emb, p_emb, w1, b1, w2, b2)
    return out.reshape(B)


def _edge_u(h_full, as_v, ad_v, ei):
    h2 = jnp.stack([h_full[:, :HH], h_full[:, HH:]])
    src3d = ei[0].reshape(NS, 5, -1, EK)
    dst3d = ei[1].reshape(NS, 5, -1, EK)
    return _edge_sc(h2, as_v, ad_v, src3d, dst3d)


def _wcat(pa, pb, pda, pdb):
    cols = jnp.stack([
        pa['W_src'] @ pa['att_src'],
        pb['W_src'] @ pb['att_src'],
        pda['W_dst'] @ pda['att_dst'],
        pdb['W_dst'] @ pdb['att_dst'],
    ], axis=1)
    pad = jnp.zeros((H, 124), jnp.float32)
    return jnp.concatenate([pa['W_src'], pb['W_src'], cols, pad], axis=1)


def kernel(x_metabolite, x_protein, params, edge_index_pp, edge_index_mm,
           edge_index_mp, edge_index_pm, metabolite_idx, protein_idx):
    h_m = _matmul(x_metabolite, params['proj_m']['W'], params['proj_m']['b'])
    h_p = _matmul(x_protein, params['proj_p']['W'], params['proj_p']['b'])
    z384 = jnp.zeros((384,), jnp.float32)
    for lp in params['layers']:
        y_p = _matmul(h_p, _wcat(lp['pp'], lp['pm'], lp['pp'], lp['mp']), z384)
        y_m = _matmul(h_m, _wcat(lp['mm'], lp['mp'], lp['mm'], lp['pm']), z384)
        u_pp = _edge_u(y_p[:, :H], y_p[:, 256], y_p[:, 258], edge_index_pp)
        u_mp = _edge_u(y_m[:, H:2 * H], y_m[:, 257], y_p[:, 259], edge_index_mp)
        u_mm = _edge_u(y_m[:, :H], y_m[:, 256], y_m[:, 258], edge_index_mm)
        u_pm = _edge_u(y_p[:, H:2 * H], y_p[:, 257], y_m[:, 259], edge_index_pm)
        h_p = _combine_bn(u_pp, u_mp, lp['pp']['bias'], lp['mp']['bias'],
                          lp['bn_p']['g'], lp['bn_p']['b'])
        h_m = _combine_bn(u_mm, u_pm, lp['mm']['bias'], lp['pm']['bias'],
                          lp['bn_m']['g'], lp['bn_m']['b'])
    mi3 = metabolite_idx.reshape(NW, -1, 128)
    pi3 = protein_idx.reshape(NW, -1, 128)
    m_emb, p_emb = _pair_gather_sc(h_m, h_p, mi3, pi3)
    return _mlp_head(m_emb, p_emb, params['mlp'])
